# baseline (speedup 1.0000x reference)
import functools

import jax
import jax.numpy as jnp
from jax.experimental import pallas as pl
from jax.experimental.pallas import tpu as pltpu

MASKB = float(2 << 19)
EPS = 1e-5
NG = 2


def _leaky(v):
    return jnp.where(v >= 0, v, 0.2 * v)


def _dist_kernel(xa_ref, ya_ref, xb_ref, yb_ref, sqx_ref, sqy_ref, d2_ref):
    dot = jnp.dot(xa_ref[...], ya_ref[...],
                  preferred_element_type=jnp.float32)
    sq = sqx_ref[...] + sqy_ref[0:1, :]
    neq = xb_ref[...] != yb_ref[0:1, :]
    d2_ref[...] = sq - 2.0 * dot + jnp.where(neq, MASKB, 0.0)


def _ab_kernel(out_ref, w_ref, b1_ref, a_ref, b_ref):
    ab = jnp.dot(out_ref[...], w_ref[...], preferred_element_type=jnp.float32,
                 precision=jax.lax.Precision.HIGHEST)
    wp = a_ref.shape[-1]
    a_ref[...] = ab[:, :wp] + b1_ref[...]
    b_ref[...] = ab[:, wp:]


def _agg_kernel(a_ref, bg_ref, d2s_ref, wd_ref, w2_ref, b2_ref, g_ref,
                bt_ref, prev_ref, out_ref, *, nk):
    rb = a_ref.shape[0]
    wp = a_ref.shape[1]
    bg = bg_ref[...].reshape(rb, nk, wp)
    feat = (a_ref[...][:, None, :] + bg
            + d2s_ref[...][:, :, None] * wd_ref[...][None, :, :])
    s = jnp.sum(_leaky(feat), axis=1)
    msg = (jnp.dot(s, w2_ref[...], preferred_element_type=jnp.float32,
                   precision=jax.lax.Precision.HIGHEST)
           + float(nk) * b2_ref[...])
    d = msg.shape[1]
    g = d // NG
    lane = jax.lax.broadcasted_iota(jnp.int32, msg.shape, 1)
    in0 = lane < g
    m0 = jnp.sum(jnp.where(in0, msg, 0.0), axis=1, keepdims=True) / g
    m1 = jnp.sum(jnp.where(in0, 0.0, msg), axis=1, keepdims=True) / g
    mean = jnp.where(in0, m0, m1)
    dev = msg - mean
    v0 = jnp.sum(jnp.where(in0, dev * dev, 0.0), axis=1, keepdims=True) / g
    v1 = jnp.sum(jnp.where(in0, 0.0, dev * dev), axis=1, keepdims=True) / g
    var = jnp.where(in0, v0, v1)
    xn = dev / jnp.sqrt(var + EPS)
    gn = xn * g_ref[...] + bt_ref[...]
    out_ref[...] = prev_ref[...] + _leaky(gn)


def kernel(x, y, y_atomtypes, x_batch, y_batch, W1, b1, W2, b2, gamma, beta):
    n, d = y_atomtypes.shape
    kk = 17
    nk = kk - 1
    nl, h, _ = W1.shape
    wp = ((h + 15) // 16) * 16
    rb = 512
    np_ = ((n + rb - 1) // rb) * rb
    cp = ((n + 127) // 128) * 128

    xb = x_batch.astype(jnp.float32)
    yb = y_batch.astype(jnp.float32)
    sqx = jnp.sum(x * x, axis=1)
    sqy = jnp.sum(y * y, axis=1)
    one = jnp.ones((n,), jnp.float32)

    zero = jnp.zeros((n,), jnp.float32)
    xa = jnp.stack([x[:, 0], x[:, 1], x[:, 2],
                    zero, zero, zero, zero, zero], axis=1)
    ya = jnp.stack([y[:, 0], y[:, 1], y[:, 2],
                    zero, zero, zero, zero, zero], axis=1)
    xa = jnp.zeros((np_, 8), jnp.float32).at[:n].set(xa)
    yap = jnp.zeros((cp, 8), jnp.float32).at[:n].set(ya)
    yat = yap.T
    xbf = jnp.zeros((np_, 1), jnp.float32).at[:n, 0].set(xb)
    ybf = jnp.broadcast_to(
        jnp.full((cp,), -1.0, jnp.float32).at[:n].set(yb), (8, cp))
    sqxc = jnp.zeros((np_, 1), jnp.float32).at[:n, 0].set(sqx)
    sqyr = jnp.broadcast_to(
        jnp.zeros((cp,), jnp.float32).at[:n].set(sqy), (8, cp))

    d2 = pl.pallas_call(
        _dist_kernel,
        grid=(np_ // rb,),
        in_specs=[pl.BlockSpec((rb, 8), lambda i: (i, 0)),
                  pl.BlockSpec((8, cp), lambda i: (0, 0)),
                  pl.BlockSpec((rb, 1), lambda i: (i, 0)),
                  pl.BlockSpec((8, cp), lambda i: (0, 0)),
                  pl.BlockSpec((rb, 1), lambda i: (i, 0)),
                  pl.BlockSpec((8, cp), lambda i: (0, 0))],
        out_specs=pl.BlockSpec((rb, cp), lambda i: (i, 0)),
        out_shape=jax.ShapeDtypeStruct((np_, cp), jnp.float32),
    )(xa, yat, xbf, ybf, sqxc, sqyr)

    negv, idx = jax.lax.top_k(-d2[:n], kk)
    idx2 = idx[:, 1:]
    y_ik = jnp.take(y, idx2.reshape(-1), axis=0).reshape(n, nk, 3)
    d2s = jnp.sum((x[:, None, :] - y_ik) ** 2, axis=-1)
    idx2 = jnp.zeros((np_, nk), jnp.int32).at[:n].set(idx2)
    d2s = jnp.zeros((np_, nk), jnp.float32).at[:n].set(d2s)

    if True:
        return d2s + idx2.astype(jnp.float32)
    out = jnp.zeros((np_, d), jnp.float32).at[:n].set(y_atomtypes)
    arb = 256
    flat_idx = idx2.reshape(-1)

    for i in range(nl):
        w1cat = jnp.zeros((d, 2 * wp), jnp.float32)
        w1cat = w1cat.at[:, :h].set(W1[i][:d, :])
        w1cat = w1cat.at[:, wp:wp + h].set(W1[i][d:2 * d, :])
        b1p = jnp.zeros((1, wp), jnp.float32).at[0, :h].set(b1[i])
        wdp = jnp.zeros((1, wp), jnp.float32).at[0, :h].set(W1[i][2 * d, :])
        w2p = jnp.zeros((wp, d), jnp.float32).at[:h, :].set(W2[i])

        a_arr, b_arr = pl.pallas_call(
            _ab_kernel,
            grid=(np_ // rb,),
            in_specs=[pl.BlockSpec((rb, d), lambda i_: (i_, 0)),
                      pl.BlockSpec((d, 2 * wp), lambda i_: (0, 0)),
                      pl.BlockSpec((1, wp), lambda i_: (0, 0))],
            out_specs=[pl.BlockSpec((rb, wp), lambda i_: (i_, 0)),
                       pl.BlockSpec((rb, wp), lambda i_: (i_, 0))],
            out_shape=[jax.ShapeDtypeStruct((np_, wp), jnp.float32),
                       jax.ShapeDtypeStruct((np_, wp), jnp.float32)],
        )(out, w1cat, b1p)

        bg = jnp.take(b_arr, flat_idx, axis=0)

        out = pl.pallas_call(
            functools.partial(_agg_kernel, nk=nk),
            grid=(np_ // arb,),
            in_specs=[pl.BlockSpec((arb, wp), lambda i_: (i_, 0)),
                      pl.BlockSpec((arb * nk, wp), lambda i_: (i_, 0)),
                      pl.BlockSpec((arb, nk), lambda i_: (i_, 0)),
                      pl.BlockSpec((1, wp), lambda i_: (0, 0)),
                      pl.BlockSpec((wp, d), lambda i_: (0, 0)),
                      pl.BlockSpec((1, d), lambda i_: (0, 0)),
                      pl.BlockSpec((1, d), lambda i_: (0, 0)),
                      pl.BlockSpec((1, d), lambda i_: (0, 0)),
                      pl.BlockSpec((arb, d), lambda i_: (i_, 0))],
            out_specs=pl.BlockSpec((arb, d), lambda i_: (i_, 0)),
            out_shape=jax.ShapeDtypeStruct((np_, d), jnp.float32),
        )(a_arr, bg, d2s, wdp, w2p,
          b2[i][None, :], gamma[i][None, :], beta[i][None, :], out)

    return out[:n]


# speedup vs baseline: 31.0902x; 31.0902x over previous
import dataclasses
import functools

import jax
import jax.numpy as jnp
from jax import lax
from jax.experimental import pallas as pl
from jax.experimental.pallas import tpu as pltpu
from jax.experimental.pallas import tpu_sc as plsc

MASKB = float(2 << 19)
EPS = 1e-5
NG = 2
BIGF = 3.0e38

_NC, _NS, _L = 2, 16, 16
_NW = _NC * _NS


def _leaky(v):
    return jnp.where(v >= 0, v, 0.2 * v)


def _dist_kernel(xa_ref, ya_ref, xb_ref, yb_ref, sqx_ref, sqy_ref,
                 d2_ref, bm_ref, *, kk):
    dot = jnp.dot(xa_ref[...], ya_ref[...],
                  preferred_element_type=jnp.float32)
    sq = sqx_ref[...] + sqy_ref[0:1, :]
    neq = xb_ref[...] != yb_ref[0:1, :]
    dd = sq - 2.0 * dot + jnp.where(neq, MASKB, 0.0)
    d2_ref[...] = dd

    rb, cp = dd.shape
    ns = cp // 128
    dv = dd.reshape(rb, ns, 128)
    lm = jnp.min(dv, axis=1)
    for _ in range(kk - 1):
        m = jnp.min(lm, axis=1, keepdims=True)
        lm = jnp.where(lm == m, BIGF, lm)
    tau = jnp.min(lm, axis=1, keepdims=True)

    ns_pad = ((ns + 15) // 16) * 16
    for g in range(ns_pad // 16):
        acc = jnp.zeros((rb, 128), jnp.float32)
        for j in range(16):
            sidx = g * 16 + j
            if sidx < ns:
                acc = acc + jnp.where(dv[:, sidx, :] <= tau,
                                      float(1 << j), 0.0)
        bm_ref[:, g * 128:(g + 1) * 128] = acc


def _make_select(np_, cp, nwords, nk):
    rows_per_w = np_ // _NW
    ch = 8
    nch = rows_per_w // ch
    ncap = 48
    wcap = 80
    nwc = nwords // 16
    gpr = cp // 128
    mesh = plsc.VectorSubcoreMesh(core_axis_name="c", subcore_axis_name="s")
    cparams = pltpu.CompilerParams()
    if "needs_layout_passes" in pltpu.CompilerParams.__dataclass_fields__:
        cparams = dataclasses.replace(cparams, needs_layout_passes=False)
    if "use_tc_tiling_on_sc" in pltpu.CompilerParams.__dataclass_fields__:
        cparams = dataclasses.replace(cparams, use_tc_tiling_on_sc=False)

    def body(bm_hbm, d2g_hbm, y_hbm, x_hbm, idx_hbm, dst_hbm,
             bmc, xc, wval, widx, cand, gidx, ncb, grows, ygath,
             oidx, odst, tmpk, tmpv, sem_g, sem_y):
        w = lax.axis_index("s") * _NC + lax.axis_index("c")
        row0 = w * rows_per_w
        iota = lax.iota(jnp.int32, _L)
        zi = jnp.zeros((_L,), jnp.int32)

        @pl.loop(0, ch)
        def _init(i):
            for b in range(ncap // 16):
                cand[i, pl.ds(16 * b, 16)] = zi

        @pl.loop(0, nch)
        def _chunk(ci):
            r0 = row0 + ci * ch
            pltpu.sync_copy(bm_hbm.at[pl.ds(r0, ch)], bmc)
            pltpu.sync_copy(x_hbm.at[pl.ds(r0, ch)], xc)

            @pl.loop(0, ch)
            def _l1(i):
                r = r0 + i
                nw = 0
                for wc in range(nwc):
                    wv = bmc[i, pl.ds(wc * 16, 16)]
                    m = wv != 0.0
                    nws = jnp.minimum(nw, wcap - 16)
                    plsc.store_compressed(wval.at[pl.ds(nws, 16)], wv, mask=m)
                    plsc.store_compressed(
                        widx.at[pl.ds(nws, 16)],
                        jnp.full((_L,), wc * 16, jnp.int32) + iota, mask=m)
                    nw = nw + jnp.sum(m.astype(jnp.int32))

                def decw(j, nc):
                    jj = jnp.full((_L,), j, jnp.int32)
                    wvi = plsc.load_gather(wval, [jj]).astype(jnp.int32)
                    wj = plsc.load_gather(widx, [jj])
                    msk = ((wvi >> iota) & 1) == 1
                    cols = ((wj // 128) * 16 + iota) * 128 + (wj % 128)
                    ncs = jnp.minimum(nc, ncap - 16)
                    plsc.store_compressed(cand.at[i].at[pl.ds(ncs, 16)],
                                          cols, mask=msk)
                    return nc + jnp.sum(msk.astype(jnp.int32))

                nc = lax.fori_loop(0, jnp.minimum(nw, wcap - 16), decw, 0)
                ncb[i, pl.ds(0, 16)] = jnp.full((_L,), nc, jnp.int32)
                for b in range(ncap // 16):
                    cv = cand[i, pl.ds(16 * b, 16)]
                    gidx[i, pl.ds(16 * b, 16)] = r * gpr + cv // 128
                pltpu.async_copy(d2g_hbm.at[gidx.at[i]], grows.at[i], sem_g)

            @pl.loop(0, ch)
            def _d1(i):
                pltpu.make_async_copy(
                    d2g_hbm.at[gidx.at[i]], grows.at[i], sem_g).wait()

            @pl.loop(0, ch)
            def _l2(i):
                ncv = ncb[i, pl.ds(0, 16)]
                ks, vs = [], []
                for b in range(ncap // 16):
                    cv = cand[i, pl.ds(16 * b, 16)]
                    ri = iota + 16 * b
                    vals = plsc.load_gather(grows.at[i], [ri, cv % 128])
                    vals = jnp.where(ri < ncv, vals, BIGF)
                    sk, sv = plsc.sort_key_val(vals, cv)
                    ks.append(sk)
                    vs.append(sv)

                def merge(ka, va, kb, vb):
                    rk = lax.rev(kb, (0,))
                    rv = lax.rev(vb, (0,))
                    sel = ka <= rk
                    lk, lv = plsc.sort_key_val(
                        jnp.minimum(ka, rk), jnp.where(sel, va, rv))
                    hk, hv = plsc.sort_key_val(
                        jnp.maximum(ka, rk), jnp.where(sel, rv, va))
                    return lk, lv, hk, hv

                l1k, l1v, h1k, h1v = merge(ks[0], vs[0], ks[1], vs[1])
                l2k, l2v, h2k, h2v = merge(l1k, l1v, ks[2], vs[2])
                is0 = iota == 0
                m1 = jnp.min(jnp.where(is0, h1k, BIGF))
                m2 = jnp.min(jnp.where(is0, h2k, BIGF))
                c1 = jnp.max(jnp.where(is0, h1v, -1))
                c2 = jnp.max(jnp.where(is0, h2v, -1))
                k17 = jnp.where(m2 <= m1, m2, m1)
                c17 = jnp.where(m2 <= m1, c2, c1)
                tmpk[pl.ds(0, 16)] = l2k
                tmpv[pl.ds(0, 16)] = l2v
                sh = jnp.minimum(iota + 1, 15)
                fk = jnp.where(iota == 15, k17, plsc.load_gather(tmpk, [sh]))
                fv = jnp.where(iota == 15, c17, plsc.load_gather(tmpv, [sh]))
                oidx[i, pl.ds(0, 16)] = fv
                odst[i, pl.ds(0, 16)] = fk
                pltpu.async_copy(y_hbm.at[fv], ygath.at[i], sem_y)

            @pl.loop(0, ch)
            def _d2l(i):
                pltpu.make_async_copy(
                    y_hbm.at[oidx.at[i]], ygath.at[i], sem_y).wait()

            @pl.loop(0, ch)
            def _l3(i):
                ii = jnp.full((_L,), i, jnp.int32)
                acc = None
                for c in range(3):
                    cc = jnp.full((_L,), c, jnp.int32)
                    xcb = plsc.load_gather(xc, [ii, cc])
                    yc = plsc.load_gather(ygath, [ii, iota, cc])
                    delta = xcb - yc
                    sqd = delta * delta
                    acc = sqd if acc is None else acc + sqd
                odst[i, pl.ds(0, 16)] = acc

            pltpu.sync_copy(oidx, idx_hbm.at[pl.ds(r0, ch)])
            pltpu.sync_copy(odst, dst_hbm.at[pl.ds(r0, ch)])

    return pl.kernel(
        body,
        out_type=[jax.ShapeDtypeStruct((np_, 16), jnp.int32),
                  jax.ShapeDtypeStruct((np_, 16), jnp.float32)],
        mesh=mesh,
        scratch_types=[
            pltpu.VMEM((ch, nwords), jnp.float32),
            pltpu.VMEM((ch, 16), jnp.float32),
            pltpu.VMEM((wcap,), jnp.float32),
            pltpu.VMEM((wcap,), jnp.int32),
            pltpu.VMEM((ch, ncap), jnp.int32),
            pltpu.VMEM((ch, ncap), jnp.int32),
            pltpu.VMEM((ch, 16), jnp.int32),
            pltpu.VMEM((ch, ncap, 128), jnp.float32),
            pltpu.VMEM((ch, 16, 128), jnp.float32),
            pltpu.VMEM((ch, 16), jnp.int32),
            pltpu.VMEM((ch, 16), jnp.float32),
            pltpu.VMEM((16,), jnp.float32),
            pltpu.VMEM((16,), jnp.int32),
            pltpu.SemaphoreType.DMA,
            pltpu.SemaphoreType.DMA,
        ],
        compiler_params=cparams)


def _make_gather(e, wp):
    per_w = e // _NW
    chg = 128
    nch = per_w // chg
    mesh = plsc.VectorSubcoreMesh(core_axis_name="c", subcore_axis_name="s")

    def body(t_hbm, i_hbm, o_hbm, idxv, rows, semg):
        w = lax.axis_index("s") * _NC + lax.axis_index("c")
        base = w * per_w
        pltpu.sync_copy(i_hbm.at[pl.ds(base, per_w)], idxv)

        @pl.loop(0, nch)
        def _c(j):
            off = j * chg
            pltpu.async_copy(
                t_hbm.at[idxv.at[pl.ds(off, chg)]], rows, semg).wait()
            pltpu.sync_copy(rows, o_hbm.at[pl.ds(base + off, chg)])

    return pl.kernel(
        body,
        out_type=jax.ShapeDtypeStruct((e, wp), jnp.float32),
        mesh=mesh,
        scratch_types=[
            pltpu.VMEM((per_w,), jnp.int32),
            pltpu.VMEM((chg, wp), jnp.float32),
            pltpu.SemaphoreType.DMA,
        ])


def _ab_kernel(out_ref, w_ref, b1_ref, a_ref, b_ref):
    ab = jnp.dot(out_ref[...], w_ref[...], preferred_element_type=jnp.float32,
                 precision=jax.lax.Precision.HIGHEST)
    wp = a_ref.shape[-1]
    a_ref[...] = ab[:, :wp] + b1_ref[...]
    b_ref[...] = ab[:, wp:]


def _agg_kernel(a_ref, bg_ref, d2s_ref, wd_ref, w2_ref, b2_ref, g_ref,
                bt_ref, prev_ref, out_ref, *, nk):
    rb = a_ref.shape[0]
    wp = a_ref.shape[1]
    bg = bg_ref[...].reshape(rb, nk, wp)
    feat = (a_ref[...][:, None, :] + bg
            + d2s_ref[...][:, :, None] * wd_ref[...][None, :, :])
    s = jnp.sum(_leaky(feat), axis=1)
    msg = (jnp.dot(s, w2_ref[...], preferred_element_type=jnp.float32,
                   precision=jax.lax.Precision.HIGHEST)
           + float(nk) * b2_ref[...])
    d = msg.shape[1]
    g = d // NG
    lane = jax.lax.broadcasted_iota(jnp.int32, msg.shape, 1)
    in0 = lane < g
    m0 = jnp.sum(jnp.where(in0, msg, 0.0), axis=1, keepdims=True) / g
    m1 = jnp.sum(jnp.where(in0, 0.0, msg), axis=1, keepdims=True) / g
    mean = jnp.where(in0, m0, m1)
    dev = msg - mean
    v0 = jnp.sum(jnp.where(in0, dev * dev, 0.0), axis=1, keepdims=True) / g
    v1 = jnp.sum(jnp.where(in0, 0.0, dev * dev), axis=1, keepdims=True) / g
    var = jnp.where(in0, v0, v1)
    xn = dev / jnp.sqrt(var + EPS)
    gn = xn * g_ref[...] + bt_ref[...]
    out_ref[...] = prev_ref[...] + _leaky(gn)


def kernel(x, y, y_atomtypes, x_batch, y_batch, W1, b1, W2, b2, gamma, beta):
    n, d = y_atomtypes.shape
    kk = 17
    nk = kk - 1
    nl, h, _ = W1.shape
    wp = ((h + 127) // 128) * 128
    rb = 256
    np_ = ((n + 511) // 512) * 512
    cp = ((n + 127) // 128) * 128
    nwords = (((cp // 128) + 15) // 16) * 16 * 8

    xb = x_batch.astype(jnp.float32)
    yb = y_batch.astype(jnp.float32)
    sqx = jnp.sum(x * x, axis=1)
    sqy = jnp.sum(y * y, axis=1)

    zero = jnp.zeros((n,), jnp.float32)
    xa = jnp.stack([x[:, 0], x[:, 1], x[:, 2],
                    zero, zero, zero, zero, zero], axis=1)
    ya = jnp.stack([y[:, 0], y[:, 1], y[:, 2],
                    zero, zero, zero, zero, zero], axis=1)
    xa = jnp.zeros((np_, 8), jnp.float32).at[:n].set(xa)
    yap = jnp.zeros((cp, 8), jnp.float32).at[:n].set(ya)
    yat = yap.T
    xbf = jnp.zeros((np_, 1), jnp.float32).at[:n, 0].set(xb)
    ybf = jnp.broadcast_to(
        jnp.full((cp,), -1.0, jnp.float32).at[:n].set(yb), (8, cp))
    sqxc = jnp.zeros((np_, 1), jnp.float32).at[:n, 0].set(sqx)
    sqyr = jnp.broadcast_to(
        jnp.zeros((cp,), jnp.float32).at[:n].set(sqy), (8, cp))

    d2, bm = pl.pallas_call(
        functools.partial(_dist_kernel, kk=kk),
        grid=(np_ // rb,),
        in_specs=[pl.BlockSpec((rb, 8), lambda i: (i, 0)),
                  pl.BlockSpec((8, cp), lambda i: (0, 0)),
                  pl.BlockSpec((rb, 1), lambda i: (i, 0)),
                  pl.BlockSpec((8, cp), lambda i: (0, 0)),
                  pl.BlockSpec((rb, 1), lambda i: (i, 0)),
                  pl.BlockSpec((8, cp), lambda i: (0, 0))],
        out_specs=[pl.BlockSpec((rb, cp), lambda i: (i, 0)),
                   pl.BlockSpec((rb, nwords), lambda i: (i, 0))],
        out_shape=[jax.ShapeDtypeStruct((np_, cp), jnp.float32),
                   jax.ShapeDtypeStruct((np_, nwords), jnp.float32)],
    )(xa, yat, xbf, ybf, sqxc, sqyr)

    d2g = d2.reshape(np_ * (cp // 128), 128)
    ypadg = jnp.zeros((cp, 128), jnp.float32).at[:n, :3].set(y)
    xpadg = jnp.zeros((np_, 16), jnp.float32).at[:n, :3].set(x)
    idx2, d2s = _make_select(np_, cp, nwords, nk)(bm, d2g, ypadg, xpadg)

    out = jnp.zeros((np_, d), jnp.float32).at[:n].set(y_atomtypes)
    arb = 256
    flat_idx = idx2.reshape(-1)
    e = np_ * nk

    for i in range(nl):
        w1cat = jnp.zeros((d, 2 * wp), jnp.float32)
        w1cat = w1cat.at[:, :h].set(W1[i][:d, :])
        w1cat = w1cat.at[:, wp:wp + h].set(W1[i][d:2 * d, :])
        b1p = jnp.zeros((1, wp), jnp.float32).at[0, :h].set(b1[i])
        wdp = jnp.zeros((1, wp), jnp.float32).at[0, :h].set(W1[i][2 * d, :])
        w2p = jnp.zeros((wp, d), jnp.float32).at[:h, :].set(W2[i])

        a_arr, b_arr = pl.pallas_call(
            _ab_kernel,
            grid=(np_ // rb,),
            in_specs=[pl.BlockSpec((rb, d), lambda i_: (i_, 0)),
                      pl.BlockSpec((d, 2 * wp), lambda i_: (0, 0)),
                      pl.BlockSpec((1, wp), lambda i_: (0, 0))],
            out_specs=[pl.BlockSpec((rb, wp), lambda i_: (i_, 0)),
                       pl.BlockSpec((rb, wp), lambda i_: (i_, 0))],
            out_shape=[jax.ShapeDtypeStruct((np_, wp), jnp.float32),
                       jax.ShapeDtypeStruct((np_, wp), jnp.float32)],
        )(out, w1cat, b1p)

        bg = _make_gather(e, wp)(b_arr, flat_idx)

        out = pl.pallas_call(
            functools.partial(_agg_kernel, nk=nk),
            grid=(np_ // arb,),
            in_specs=[pl.BlockSpec((arb, wp), lambda i_: (i_, 0)),
                      pl.BlockSpec((arb * nk, wp), lambda i_: (i_, 0)),
                      pl.BlockSpec((arb, nk), lambda i_: (i_, 0)),
                      pl.BlockSpec((1, wp), lambda i_: (0, 0)),
                      pl.BlockSpec((wp, d), lambda i_: (0, 0)),
                      pl.BlockSpec((1, d), lambda i_: (0, 0)),
                      pl.BlockSpec((1, d), lambda i_: (0, 0)),
                      pl.BlockSpec((1, d), lambda i_: (0, 0)),
                      pl.BlockSpec((arb, d), lambda i_: (i_, 0))],
            out_specs=pl.BlockSpec((arb, d), lambda i_: (i_, 0)),
            out_shape=jax.ShapeDtypeStruct((np_, d), jnp.float32),
        )(a_arr, bg, d2s, wdp, w2p,
          b2[i][None, :], gamma[i][None, :], beta[i][None, :], out)

    return out[:n]


# speedup vs baseline: 33.2750x; 1.0703x over previous
import dataclasses
import functools

import jax
import jax.numpy as jnp
from jax import lax
from jax.experimental import pallas as pl
from jax.experimental.pallas import tpu as pltpu
from jax.experimental.pallas import tpu_sc as plsc

MASKB = float(2 << 19)
EPS = 1e-5
NG = 2
BIGF = 3.0e38

_NC, _NS, _L = 2, 16, 16
_NW = _NC * _NS


def _leaky(v):
    return jnp.where(v >= 0, v, 0.2 * v)


def _dist_kernel(xa_ref, ya_ref, xb_ref, yb_ref, sqx_ref, sqy_ref,
                 d2_ref, bm_ref, *, kk):
    dot = jnp.dot(xa_ref[...], ya_ref[...],
                  preferred_element_type=jnp.float32)
    sq = sqx_ref[...] + sqy_ref[0:1, :]
    neq = xb_ref[...] != yb_ref[0:1, :]
    dd = sq - 2.0 * dot + jnp.where(neq, MASKB, 0.0)
    d2_ref[...] = dd

    rb, cp = dd.shape
    ns = cp // 128
    dv = dd.reshape(rb, ns, 128)
    lm = jnp.min(dv, axis=1)
    for _ in range(kk - 1):
        m = jnp.min(lm, axis=1, keepdims=True)
        lm = jnp.where(lm == m, BIGF, lm)
    tau = jnp.min(lm, axis=1, keepdims=True)

    ns_pad = ((ns + 15) // 16) * 16
    for g in range(ns_pad // 16):
        acc = jnp.zeros((rb, 128), jnp.float32)
        for j in range(16):
            sidx = g * 16 + j
            if sidx < ns:
                acc = acc + jnp.where(dv[:, sidx, :] <= tau,
                                      float(1 << j), 0.0)
        bm_ref[:, g * 128:(g + 1) * 128] = acc


def _make_select(np_, cp, nwords, nk):
    rows_per_w = np_ // _NW
    ch = 16
    nch = rows_per_w // ch
    ncap = 48
    wcap = 80
    nwc = nwords // 16
    gpr = cp // 16
    mesh = plsc.VectorSubcoreMesh(core_axis_name="c", subcore_axis_name="s")
    cparams = pltpu.CompilerParams()
    if "needs_layout_passes" in pltpu.CompilerParams.__dataclass_fields__:
        cparams = dataclasses.replace(cparams, needs_layout_passes=False)
    if "use_tc_tiling_on_sc" in pltpu.CompilerParams.__dataclass_fields__:
        cparams = dataclasses.replace(cparams, use_tc_tiling_on_sc=False)

    def body(bm_hbm, d2g_hbm, y_hbm, x_hbm, idx_hbm, dst_hbm,
             bmc, xc, wval, widx, cand, gidx, ncb, grows, ygath,
             oidx, odst, tmpk, tmpv, sem_g, sem_y):
        w = lax.axis_index("s") * _NC + lax.axis_index("c")
        row0 = w * rows_per_w
        iota = lax.iota(jnp.int32, _L)
        zi = jnp.zeros((_L,), jnp.int32)

        @pl.loop(0, ch)
        def _init(i):
            for b in range(ncap // 16):
                cand[i, pl.ds(16 * b, 16)] = zi

        @pl.loop(0, nch)
        def _chunk(ci):
            r0 = row0 + ci * ch
            pltpu.sync_copy(bm_hbm.at[pl.ds(r0, ch)], bmc)
            pltpu.sync_copy(x_hbm.at[pl.ds(r0, ch)], xc)

            @pl.loop(0, ch)
            def _l1(i):
                r = r0 + i
                nw = 0
                for wc in range(nwc):
                    wv = bmc[i, pl.ds(wc * 16, 16)]
                    m = wv != 0.0
                    nws = jnp.minimum(nw, wcap - 16)
                    plsc.store_compressed(wval.at[pl.ds(nws, 16)], wv, mask=m)
                    plsc.store_compressed(
                        widx.at[pl.ds(nws, 16)],
                        jnp.full((_L,), wc * 16, jnp.int32) + iota, mask=m)
                    nw = nw + jnp.sum(m.astype(jnp.int32))

                def decw(j, nc):
                    jj = jnp.full((_L,), j, jnp.int32)
                    wvi = plsc.load_gather(wval, [jj]).astype(jnp.int32)
                    wj = plsc.load_gather(widx, [jj])
                    msk = ((wvi >> iota) & 1) == 1
                    cols = ((wj // 128) * 16 + iota) * 128 + (wj % 128)
                    ncs = jnp.minimum(nc, ncap - 16)
                    plsc.store_compressed(cand.at[i].at[pl.ds(ncs, 16)],
                                          cols, mask=msk)
                    return nc + jnp.sum(msk.astype(jnp.int32))

                nc = lax.fori_loop(0, jnp.minimum(nw, wcap - 16), decw, 0)
                ncb[i, pl.ds(0, 16)] = jnp.full((_L,), nc, jnp.int32)
                for b in range(ncap // 16):
                    cv = cand[i, pl.ds(16 * b, 16)]
                    gidx[i, pl.ds(16 * b, 16)] = r * gpr + cv // 16
                pltpu.async_copy(d2g_hbm.at[gidx.at[i]], grows.at[i], sem_g)

            @pl.loop(0, ch)
            def _d1(i):
                pltpu.make_async_copy(
                    d2g_hbm.at[gidx.at[i]], grows.at[i], sem_g).wait()

            @pl.loop(0, ch)
            def _l2(i):
                ncv = ncb[i, pl.ds(0, 16)]
                ks, vs = [], []
                for b in range(ncap // 16):
                    cv = cand[i, pl.ds(16 * b, 16)]
                    ri = iota + 16 * b
                    vals = plsc.load_gather(grows.at[i], [ri, cv % 16])
                    vals = jnp.where(ri < ncv, vals, BIGF)
                    sk, sv = plsc.sort_key_val(vals, cv)
                    ks.append(sk)
                    vs.append(sv)

                def merge(ka, va, kb, vb):
                    rk = lax.rev(kb, (0,))
                    rv = lax.rev(vb, (0,))
                    sel = ka <= rk
                    lk, lv = plsc.sort_key_val(
                        jnp.minimum(ka, rk), jnp.where(sel, va, rv))
                    hk, hv = plsc.sort_key_val(
                        jnp.maximum(ka, rk), jnp.where(sel, rv, va))
                    return lk, lv, hk, hv

                l1k, l1v, h1k, h1v = merge(ks[0], vs[0], ks[1], vs[1])
                l2k, l2v, h2k, h2v = merge(l1k, l1v, ks[2], vs[2])
                is0 = iota == 0
                m1 = jnp.min(jnp.where(is0, h1k, BIGF))
                m2 = jnp.min(jnp.where(is0, h2k, BIGF))
                c1 = jnp.max(jnp.where(is0, h1v, -1))
                c2 = jnp.max(jnp.where(is0, h2v, -1))
                k17 = jnp.where(m2 <= m1, m2, m1)
                c17 = jnp.where(m2 <= m1, c2, c1)
                tmpk[pl.ds(0, 16)] = l2k
                tmpv[pl.ds(0, 16)] = l2v
                sh = jnp.minimum(iota + 1, 15)
                fk = jnp.where(iota == 15, k17, plsc.load_gather(tmpk, [sh]))
                fv = jnp.where(iota == 15, c17, plsc.load_gather(tmpv, [sh]))
                oidx[i, pl.ds(0, 16)] = fv
                odst[i, pl.ds(0, 16)] = fk
                pltpu.async_copy(y_hbm.at[fv], ygath.at[i], sem_y)

            @pl.loop(0, ch)
            def _d2l(i):
                pltpu.make_async_copy(
                    y_hbm.at[oidx.at[i]], ygath.at[i], sem_y).wait()

            @pl.loop(0, ch)
            def _l3(i):
                ii = jnp.full((_L,), i, jnp.int32)
                acc = None
                for c in range(3):
                    cc = jnp.full((_L,), c, jnp.int32)
                    xcb = plsc.load_gather(xc, [ii, cc])
                    yc = plsc.load_gather(ygath, [ii, iota, cc])
                    delta = xcb - yc
                    sqd = delta * delta
                    acc = sqd if acc is None else acc + sqd
                odst[i, pl.ds(0, 16)] = acc

            pltpu.sync_copy(oidx, idx_hbm.at[pl.ds(r0, ch)])
            pltpu.sync_copy(odst, dst_hbm.at[pl.ds(r0, ch)])

    return pl.kernel(
        body,
        out_type=[jax.ShapeDtypeStruct((np_, 16), jnp.int32),
                  jax.ShapeDtypeStruct((np_, 16), jnp.float32)],
        mesh=mesh,
        scratch_types=[
            pltpu.VMEM((ch, nwords), jnp.float32),
            pltpu.VMEM((ch, 16), jnp.float32),
            pltpu.VMEM((wcap,), jnp.float32),
            pltpu.VMEM((wcap,), jnp.int32),
            pltpu.VMEM((ch, ncap), jnp.int32),
            pltpu.VMEM((ch, ncap), jnp.int32),
            pltpu.VMEM((ch, 16), jnp.int32),
            pltpu.VMEM((ch, ncap, 16), jnp.float32),
            pltpu.VMEM((ch, 16, 16), jnp.float32),
            pltpu.VMEM((ch, 16), jnp.int32),
            pltpu.VMEM((ch, 16), jnp.float32),
            pltpu.VMEM((16,), jnp.float32),
            pltpu.VMEM((16,), jnp.int32),
            pltpu.SemaphoreType.DMA,
            pltpu.SemaphoreType.DMA,
        ],
        compiler_params=cparams)


def _make_gather(e, wp):
    per_w = e // _NW
    chg = 128
    nch = per_w // chg
    mesh = plsc.VectorSubcoreMesh(core_axis_name="c", subcore_axis_name="s")

    def body(t_hbm, i_hbm, o_hbm, idxv, rows, semg):
        w = lax.axis_index("s") * _NC + lax.axis_index("c")
        base = w * per_w
        pltpu.sync_copy(i_hbm.at[pl.ds(base, per_w)], idxv)

        @pl.loop(0, nch)
        def _c(j):
            off = j * chg
            pltpu.async_copy(
                t_hbm.at[idxv.at[pl.ds(off, chg)]], rows, semg).wait()
            pltpu.sync_copy(rows, o_hbm.at[pl.ds(base + off, chg)])

    return pl.kernel(
        body,
        out_type=jax.ShapeDtypeStruct((e, wp), jnp.float32),
        mesh=mesh,
        scratch_types=[
            pltpu.VMEM((per_w,), jnp.int32),
            pltpu.VMEM((chg, wp), jnp.float32),
            pltpu.SemaphoreType.DMA,
        ])


def _ab_kernel(out_ref, w_ref, b1_ref, a_ref, b_ref):
    ab = jnp.dot(out_ref[...], w_ref[...], preferred_element_type=jnp.float32,
                 precision=jax.lax.Precision.HIGHEST)
    wp = a_ref.shape[-1]
    a_ref[...] = ab[:, :wp] + b1_ref[...]
    b_ref[...] = ab[:, wp:]


def _agg_kernel(a_ref, bg_ref, d2s_ref, wd_ref, w2_ref, b2_ref, g_ref,
                bt_ref, prev_ref, out_ref, *, nk):
    rb = a_ref.shape[0]
    wp = a_ref.shape[1]
    bg = bg_ref[...].reshape(rb, nk, wp)
    feat = (a_ref[...][:, None, :] + bg
            + d2s_ref[...][:, :, None] * wd_ref[...][None, :, :])
    s = jnp.sum(_leaky(feat), axis=1)
    msg = (jnp.dot(s, w2_ref[...], preferred_element_type=jnp.float32,
                   precision=jax.lax.Precision.HIGHEST)
           + float(nk) * b2_ref[...])
    d = msg.shape[1]
    g = d // NG
    lane = jax.lax.broadcasted_iota(jnp.int32, msg.shape, 1)
    in0 = lane < g
    m0 = jnp.sum(jnp.where(in0, msg, 0.0), axis=1, keepdims=True) / g
    m1 = jnp.sum(jnp.where(in0, 0.0, msg), axis=1, keepdims=True) / g
    mean = jnp.where(in0, m0, m1)
    dev = msg - mean
    v0 = jnp.sum(jnp.where(in0, dev * dev, 0.0), axis=1, keepdims=True) / g
    v1 = jnp.sum(jnp.where(in0, 0.0, dev * dev), axis=1, keepdims=True) / g
    var = jnp.where(in0, v0, v1)
    xn = dev / jnp.sqrt(var + EPS)
    gn = xn * g_ref[...] + bt_ref[...]
    out_ref[...] = prev_ref[...] + _leaky(gn)


def kernel(x, y, y_atomtypes, x_batch, y_batch, W1, b1, W2, b2, gamma, beta):
    n, d = y_atomtypes.shape
    kk = 17
    nk = kk - 1
    nl, h, _ = W1.shape
    wp = ((h + 127) // 128) * 128
    rb = 256
    np_ = ((n + 511) // 512) * 512
    cp = ((n + 127) // 128) * 128
    nwords = (((cp // 128) + 15) // 16) * 16 * 8

    xb = x_batch.astype(jnp.float32)
    yb = y_batch.astype(jnp.float32)
    sqx = jnp.sum(x * x, axis=1)
    sqy = jnp.sum(y * y, axis=1)

    zero = jnp.zeros((n,), jnp.float32)
    xa = jnp.stack([x[:, 0], x[:, 1], x[:, 2],
                    zero, zero, zero, zero, zero], axis=1)
    ya = jnp.stack([y[:, 0], y[:, 1], y[:, 2],
                    zero, zero, zero, zero, zero], axis=1)
    xa = jnp.zeros((np_, 8), jnp.float32).at[:n].set(xa)
    yap = jnp.zeros((cp, 8), jnp.float32).at[:n].set(ya)
    yat = yap.T
    xbf = jnp.zeros((np_, 1), jnp.float32).at[:n, 0].set(xb)
    ybf = jnp.broadcast_to(
        jnp.full((cp,), -1.0, jnp.float32).at[:n].set(yb), (8, cp))
    sqxc = jnp.zeros((np_, 1), jnp.float32).at[:n, 0].set(sqx)
    sqyr = jnp.broadcast_to(
        jnp.zeros((cp,), jnp.float32).at[:n].set(sqy), (8, cp))

    d2, bm = pl.pallas_call(
        functools.partial(_dist_kernel, kk=kk),
        grid=(np_ // rb,),
        in_specs=[pl.BlockSpec((rb, 8), lambda i: (i, 0)),
                  pl.BlockSpec((8, cp), lambda i: (0, 0)),
                  pl.BlockSpec((rb, 1), lambda i: (i, 0)),
                  pl.BlockSpec((8, cp), lambda i: (0, 0)),
                  pl.BlockSpec((rb, 1), lambda i: (i, 0)),
                  pl.BlockSpec((8, cp), lambda i: (0, 0))],
        out_specs=[pl.BlockSpec((rb, cp), lambda i: (i, 0)),
                   pl.BlockSpec((rb, nwords), lambda i: (i, 0))],
        out_shape=[jax.ShapeDtypeStruct((np_, cp), jnp.float32),
                   jax.ShapeDtypeStruct((np_, nwords), jnp.float32)],
    )(xa, yat, xbf, ybf, sqxc, sqyr)

    d2g = d2.reshape(np_ * (cp // 16), 16)
    ypadg = jnp.zeros((cp, 16), jnp.float32).at[:n, :3].set(y)
    xpadg = jnp.zeros((np_, 16), jnp.float32).at[:n, :3].set(x)
    idx2, d2s = _make_select(np_, cp, nwords, nk)(bm, d2g, ypadg, xpadg)

    out = jnp.zeros((np_, d), jnp.float32).at[:n].set(y_atomtypes)
    arb = 256
    flat_idx = idx2.reshape(-1)
    e = np_ * nk

    for i in range(nl):
        w1cat = jnp.zeros((d, 2 * wp), jnp.float32)
        w1cat = w1cat.at[:, :h].set(W1[i][:d, :])
        w1cat = w1cat.at[:, wp:wp + h].set(W1[i][d:2 * d, :])
        b1p = jnp.zeros((1, wp), jnp.float32).at[0, :h].set(b1[i])
        wdp = jnp.zeros((1, wp), jnp.float32).at[0, :h].set(W1[i][2 * d, :])
        w2p = jnp.zeros((wp, d), jnp.float32).at[:h, :].set(W2[i])

        a_arr, b_arr = pl.pallas_call(
            _ab_kernel,
            grid=(np_ // rb,),
            in_specs=[pl.BlockSpec((rb, d), lambda i_: (i_, 0)),
                      pl.BlockSpec((d, 2 * wp), lambda i_: (0, 0)),
                      pl.BlockSpec((1, wp), lambda i_: (0, 0))],
            out_specs=[pl.BlockSpec((rb, wp), lambda i_: (i_, 0)),
                       pl.BlockSpec((rb, wp), lambda i_: (i_, 0))],
            out_shape=[jax.ShapeDtypeStruct((np_, wp), jnp.float32),
                       jax.ShapeDtypeStruct((np_, wp), jnp.float32)],
        )(out, w1cat, b1p)

        bg = _make_gather(e, wp)(b_arr, flat_idx)

        out = pl.pallas_call(
            functools.partial(_agg_kernel, nk=nk),
            grid=(np_ // arb,),
            in_specs=[pl.BlockSpec((arb, wp), lambda i_: (i_, 0)),
                      pl.BlockSpec((arb * nk, wp), lambda i_: (i_, 0)),
                      pl.BlockSpec((arb, nk), lambda i_: (i_, 0)),
                      pl.BlockSpec((1, wp), lambda i_: (0, 0)),
                      pl.BlockSpec((wp, d), lambda i_: (0, 0)),
                      pl.BlockSpec((1, d), lambda i_: (0, 0)),
                      pl.BlockSpec((1, d), lambda i_: (0, 0)),
                      pl.BlockSpec((1, d), lambda i_: (0, 0)),
                      pl.BlockSpec((arb, d), lambda i_: (i_, 0))],
            out_specs=pl.BlockSpec((arb, d), lambda i_: (i_, 0)),
            out_shape=jax.ShapeDtypeStruct((np_, d), jnp.float32),
        )(a_arr, bg, d2s, wdp, w2p,
          b2[i][None, :], gamma[i][None, :], beta[i][None, :], out)

    return out[:n]
